# Initial kernel scaffold; baseline (speedup 1.0000x reference)
import jax
import jax.numpy as jnp
from jax.experimental import pallas as pl


def kernel(x, value_embeddings, scale):
    raise NotImplementedError("write your pallas kernel here")


# speedup vs baseline: 607.1556x; 607.1556x over previous
import functools

import jax
import jax.numpy as jnp
from jax import lax
from jax.experimental import pallas as pl
from jax.experimental.pallas import tpu as pltpu
from jax.experimental.pallas import tpu_sc as plsc

_NUM_EMB_PER_CH = 256
_IN_CHANNELS = 192
_MIN_SCALE = 2.5
_MAX_SCALE = 3.5

_ROW = 224 * 224
_CHUNK = _ROW // 4
_VECS = _CHUNK // 16

_NC = 2
_NS = 16
_NW = _NC * _NS


def _body(x_hbm, tab_hbm, scale_hbm, out_hbm, tab_v, scale_v, xin_v, out_v):
    wid = lax.axis_index("s") * _NC + lax.axis_index("c")

    pltpu.sync_copy(tab_hbm, tab_v)
    pltpu.sync_copy(scale_hbm, scale_v)

    n_rows_per_w = 768 // _NW
    n_chunks = n_rows_per_w * 4

    def chunk_body(k, _):
        row = wid * n_rows_per_w + k // 4
        col = (k % 4) * _CHUNK
        ch = row % _IN_CHANNELS
        base = row * _ROW + col

        pltpu.sync_copy(x_hbm.at[pl.ds(base, _CHUNK)], xin_v)

        ch_vec = jnp.full((16,), ch, dtype=jnp.int32)
        sv = plsc.load_gather(scale_v, [ch_vec])
        sv = jnp.minimum(jnp.maximum(sv, _MIN_SCALE), _MAX_SCALE)
        offv = jnp.full((16,), ch * _NUM_EMB_PER_CH, dtype=jnp.int32).astype(
            jnp.float32
        )

        def vec_body(i, _):
            xv = xin_v[pl.ds(i * 16, 16)]
            t = jnp.minimum(jnp.maximum(xv, -sv), sv)
            o1 = (t / sv + 1.0) / 2.0
            o4 = o1 * 255.0 + offv
            li = o4.astype(jnp.int32)
            lof = li.astype(jnp.float32)
            d = o4 - lof
            ui = li + (d > 0.0).astype(jnp.int32)
            lov = plsc.load_gather(tab_v, [li])
            upv = plsc.load_gather(tab_v, [ui])
            out_v[pl.ds(i * 16, 16)] = d * lov + (1.0 - d) * upv
            return 0

        lax.fori_loop(0, _VECS, vec_body, 0)
        pltpu.sync_copy(out_v, out_hbm.at[pl.ds(base, _CHUNK)])
        return 0

    lax.fori_loop(0, n_chunks, chunk_body, 0)


@jax.jit
def kernel(x, value_embeddings, scale):
    n = x.size
    x_flat = x.reshape(n)
    tab = value_embeddings.reshape(-1)
    sc = scale.reshape(-1)

    mesh = plsc.VectorSubcoreMesh(
        core_axis_name="c", subcore_axis_name="s", num_cores=_NC,
        num_subcores=_NS,
    )
    run = pl.kernel(
        _body,
        out_type=jax.ShapeDtypeStruct((n,), jnp.float32),
        mesh=mesh,
        scratch_types=[
            pltpu.VMEM((tab.shape[0],), jnp.float32),
            pltpu.VMEM((_IN_CHANNELS,), jnp.float32),
            pltpu.VMEM((_CHUNK,), jnp.float32),
            pltpu.VMEM((_CHUNK,), jnp.float32),
        ],
        compiler_params=pltpu.CompilerParams(needs_layout_passes=False),
    )
    out = run(x_flat, tab, sc)
    return out.reshape(x.shape)


# speedup vs baseline: 1052.3977x; 1.7333x over previous
import functools

import jax
import jax.numpy as jnp
from jax import lax
from jax.experimental import pallas as pl
from jax.experimental.pallas import tpu as pltpu
from jax.experimental.pallas import tpu_sc as plsc

_NUM_EMB_PER_CH = 256
_IN_CHANNELS = 192
_MIN_SCALE = 2.5
_MAX_SCALE = 3.5

_ROW = 224 * 224
_CHUNK = _ROW // 4
_VECS = _CHUNK // 16

_NC = 2
_NS = 16
_NW = _NC * _NS
_ROWS_PER_W = 768 // _NW
_CHUNKS_PER_W = _ROWS_PER_W * 4


def _body(x_hbm, tab_hbm, scale_hbm, out_hbm,
          tab_v, scale_v, xin0, xin1, xout0, xout1,
          isem0, isem1, osem0, osem1):
    wid = lax.axis_index("s") * _NC + lax.axis_index("c")

    pltpu.sync_copy(tab_hbm, tab_v)
    pltpu.sync_copy(scale_hbm, scale_v)

    def chunk_base(k):
        row = wid * _ROWS_PER_W + k // 4
        return row, row * _ROW + (k % 4) * _CHUNK

    def start_in(k, buf, sem):
        _, base = chunk_base(k)
        pltpu.async_copy(x_hbm.at[pl.ds(base, _CHUNK)], buf, sem)

    start_in(0, xin0, isem0)
    start_in(1, xin1, isem1)

    bufs = ((xin0, isem0, xout0, osem0), (xin1, isem1, xout1, osem1))

    def step(j, _):
        for b, (xin, isem, xout, osem) in enumerate(bufs):
            k = 2 * j + b
            row, base = chunk_base(k)
            ch = row % _IN_CHANNELS

            pltpu.make_async_copy(
                x_hbm.at[pl.ds(base, _CHUNK)], xin, isem).wait()

            @pl.when(j > 0)
            def _wait_out():
                pltpu.make_async_copy(
                    xout, out_hbm.at[pl.ds(base, _CHUNK)], osem).wait()

            ch_vec = jnp.full((16,), ch, dtype=jnp.int32)
            sv = plsc.load_gather(scale_v, [ch_vec])
            sv = jnp.minimum(jnp.maximum(sv, _MIN_SCALE), _MAX_SCALE)
            nsv = -sv
            offv = jnp.full(
                (16,), ch * _NUM_EMB_PER_CH, dtype=jnp.int32
            ).astype(jnp.float32)

            @plsc.parallel_loop(0, _VECS, unroll=8)
            def _vec(i):
                xv = xin[pl.ds(i * 16, 16)]
                t = jnp.minimum(jnp.maximum(xv, nsv), sv)
                o1 = (t / sv + 1.0) / 2.0
                o4 = o1 * 255.0 + offv
                li = o4.astype(jnp.int32)
                lof = li.astype(jnp.float32)
                d = o4 - lof
                ui = li + (d > 0.0).astype(jnp.int32)
                lov = plsc.load_gather(tab_v, [li])
                upv = plsc.load_gather(tab_v, [ui])
                xout[pl.ds(i * 16, 16)] = d * lov + (1.0 - d) * upv

            pltpu.async_copy(xout, out_hbm.at[pl.ds(base, _CHUNK)], osem)

            @pl.when(j < _CHUNKS_PER_W // 2 - 1)
            def _refill():
                start_in(k + 2, xin, isem)
        return 0

    lax.fori_loop(0, _CHUNKS_PER_W // 2, step, 0)

    _, last0 = chunk_base(_CHUNKS_PER_W - 2)
    _, last1 = chunk_base(_CHUNKS_PER_W - 1)
    pltpu.make_async_copy(
        xout0, out_hbm.at[pl.ds(last0, _CHUNK)], osem0).wait()
    pltpu.make_async_copy(
        xout1, out_hbm.at[pl.ds(last1, _CHUNK)], osem1).wait()


@jax.jit
def kernel(x, value_embeddings, scale):
    n = x.size
    x_flat = x.reshape(n)
    tab = value_embeddings.reshape(-1)
    sc = scale.reshape(-1)

    mesh = plsc.VectorSubcoreMesh(
        core_axis_name="c", subcore_axis_name="s", num_cores=_NC,
        num_subcores=_NS,
    )
    run = pl.kernel(
        _body,
        out_type=jax.ShapeDtypeStruct((n,), jnp.float32),
        mesh=mesh,
        scratch_types=[
            pltpu.VMEM((tab.shape[0],), jnp.float32),
            pltpu.VMEM((_IN_CHANNELS,), jnp.float32),
            pltpu.VMEM((_CHUNK,), jnp.float32),
            pltpu.VMEM((_CHUNK,), jnp.float32),
            pltpu.VMEM((_CHUNK,), jnp.float32),
            pltpu.VMEM((_CHUNK,), jnp.float32),
            pltpu.SemaphoreType.DMA,
            pltpu.SemaphoreType.DMA,
            pltpu.SemaphoreType.DMA,
            pltpu.SemaphoreType.DMA,
        ],
        compiler_params=pltpu.CompilerParams(needs_layout_passes=False),
    )
    out = run(x_flat, tab, sc)
    return out.reshape(x.shape)


# speedup vs baseline: 1058.8131x; 1.0061x over previous
import functools

import jax
import jax.numpy as jnp
from jax import lax
from jax.experimental import pallas as pl
from jax.experimental.pallas import tpu as pltpu
from jax.experimental.pallas import tpu_sc as plsc

_NUM_EMB_PER_CH = 256
_IN_CHANNELS = 192
_MIN_SCALE = 2.5
_MAX_SCALE = 3.5

_ROW = 224 * 224
_CHUNK = _ROW // 4
_VECS = _CHUNK // 16

_NC = 2
_NS = 16
_NW = _NC * _NS


def _make_body(rows_per_w):
    chunks_per_w = rows_per_w * 4

    def _body(x_hbm, tab_hbm, scale_hbm, out_hbm,
              tab_v, scale_v, xin0, xin1, xout0, xout1,
              isem0, isem1, osem0, osem1):
        wid = lax.axis_index("s") * _NC + lax.axis_index("c")

        pltpu.sync_copy(tab_hbm, tab_v)
        pltpu.sync_copy(scale_hbm, scale_v)

        def chunk_base(k):
            row = wid * rows_per_w + k // 4
            return row, row * _ROW + (k % 4) * _CHUNK

        def start_in(k, buf, sem):
            _, base = chunk_base(k)
            pltpu.async_copy(x_hbm.at[pl.ds(base, _CHUNK)], buf, sem)

        start_in(0, xin0, isem0)
        start_in(1, xin1, isem1)

        bufs = ((xin0, isem0, xout0, osem0), (xin1, isem1, xout1, osem1))

        def step(j, _):
            for b, (xin, isem, xout, osem) in enumerate(bufs):
                k = 2 * j + b
                row, base = chunk_base(k)
                ch = row % _IN_CHANNELS

                pltpu.make_async_copy(
                    x_hbm.at[pl.ds(base, _CHUNK)], xin, isem).wait()

                @pl.when(j > 0)
                def _wait_out():
                    pltpu.make_async_copy(
                        xout, out_hbm.at[pl.ds(base, _CHUNK)], osem).wait()

                ch_vec = jnp.full((16,), ch, dtype=jnp.int32)
                sv = plsc.load_gather(scale_v, [ch_vec])
                sv = jnp.minimum(jnp.maximum(sv, _MIN_SCALE), _MAX_SCALE)
                nsv = -sv
                offv = jnp.full(
                    (16,), ch * _NUM_EMB_PER_CH, dtype=jnp.int32
                ).astype(jnp.float32)

                @plsc.parallel_loop(0, _VECS, unroll=8)
                def _vec(i):
                    xv = xin[pl.ds(i * 16, 16)]
                    t = jnp.minimum(jnp.maximum(xv, nsv), sv)
                    o1 = (t / sv + 1.0) / 2.0
                    o4 = o1 * 255.0 + offv
                    li = o4.astype(jnp.int32)
                    lof = li.astype(jnp.float32)
                    d = o4 - lof
                    ui = li + (d > 0.0).astype(jnp.int32)
                    lov = plsc.load_gather(tab_v, [li])
                    upv = plsc.load_gather(tab_v, [ui])
                    xout[pl.ds(i * 16, 16)] = d * lov + (1.0 - d) * upv

                pltpu.async_copy(xout, out_hbm.at[pl.ds(base, _CHUNK)], osem)

                @pl.when(j < chunks_per_w // 2 - 1)
                def _refill():
                    start_in(k + 2, xin, isem)
            return 0

        lax.fori_loop(0, chunks_per_w // 2, step, 0)

        _, last0 = chunk_base(chunks_per_w - 2)
        _, last1 = chunk_base(chunks_per_w - 1)
        pltpu.make_async_copy(
            xout0, out_hbm.at[pl.ds(last0, _CHUNK)], osem0).wait()
        pltpu.make_async_copy(
            xout1, out_hbm.at[pl.ds(last1, _CHUNK)], osem1).wait()

    return _body


@jax.jit
def kernel(x, value_embeddings, scale):
    tab = value_embeddings.reshape(-1)
    sc = scale.reshape(-1)
    nslice = x.shape[0]
    rows = x.shape[1]
    n = rows * _ROW
    rows_per_w = rows // _NW

    mesh = plsc.VectorSubcoreMesh(
        core_axis_name="c", subcore_axis_name="s", num_cores=_NC,
        num_subcores=_NS,
    )
    run = pl.kernel(
        _make_body(rows_per_w),
        out_type=jax.ShapeDtypeStruct((n,), jnp.float32),
        mesh=mesh,
        scratch_types=[
            pltpu.VMEM((tab.shape[0],), jnp.float32),
            pltpu.VMEM((_IN_CHANNELS,), jnp.float32),
            pltpu.VMEM((_CHUNK,), jnp.float32),
            pltpu.VMEM((_CHUNK,), jnp.float32),
            pltpu.VMEM((_CHUNK,), jnp.float32),
            pltpu.VMEM((_CHUNK,), jnp.float32),
            pltpu.SemaphoreType.DMA,
            pltpu.SemaphoreType.DMA,
            pltpu.SemaphoreType.DMA,
            pltpu.SemaphoreType.DMA,
        ],
        compiler_params=pltpu.CompilerParams(needs_layout_passes=False),
    )

    outs = []
    for i in range(nslice):
        xi = x[i].reshape(n)
        outs.append(run(xi, tab, sc).reshape(1, rows, 224, 224))
    return jnp.concatenate(outs, axis=0)


# speedup vs baseline: 2165.3072x; 2.0450x over previous
import functools

import jax
import jax.numpy as jnp
from jax import lax
from jax.experimental import pallas as pl
from jax.experimental.pallas import tpu as pltpu
from jax.experimental.pallas import tpu_sc as plsc

_NUM_EMB_PER_CH = 256
_IN_CHANNELS = 192
_MIN_SCALE = 2.5
_MAX_SCALE = 3.5

_H = 224
_W = 224
_BH = 56
_WVECS = _W // 16

_NC = 2
_NS = 16
_NW = _NC * _NS
_PLANES_PER_W = 768 // _NW
_BLOCKS_PER_W = _PLANES_PER_W * 4


def _body(x_hbm, tab_hbm, scale_hbm, out_hbm,
          tab_v, scale_v, xin0, xin1, xout0, xout1,
          isem0, isem1, osem0, osem1):
    wid = lax.axis_index("s") * _NC + lax.axis_index("c")

    pltpu.sync_copy(tab_hbm, tab_v)
    pltpu.sync_copy(scale_hbm, scale_v)

    def block_loc(k):
        plane = wid * _PLANES_PER_W + k // 4
        return plane, (k % 4) * _BH

    def start_in(k, buf, sem):
        plane, h0 = block_loc(k)
        pltpu.async_copy(x_hbm.at[plane, pl.ds(h0, _BH)], buf, sem)

    start_in(0, xin0, isem0)
    start_in(1, xin1, isem1)

    bufs = ((xin0, isem0, xout0, osem0), (xin1, isem1, xout1, osem1))

    def step(j, _):
        for b, (xin, isem, xout, osem) in enumerate(bufs):
            k = 2 * j + b
            plane, h0 = block_loc(k)
            ch = plane % _IN_CHANNELS

            pltpu.make_async_copy(
                x_hbm.at[plane, pl.ds(h0, _BH)], xin, isem).wait()

            @pl.when(j > 0)
            def _wait_out():
                pltpu.make_async_copy(
                    xout, out_hbm.at[plane, pl.ds(h0, _BH)], osem).wait()

            ch_vec = jnp.full((16,), ch, dtype=jnp.int32)
            sv = plsc.load_gather(scale_v, [ch_vec])
            sv = jnp.minimum(jnp.maximum(sv, _MIN_SCALE), _MAX_SCALE)
            nsv = -sv
            offv = jnp.full(
                (16,), ch * _NUM_EMB_PER_CH, dtype=jnp.int32
            ).astype(jnp.float32)

            @plsc.parallel_loop(0, _BH, unroll=1)
            def _row(h):
                for w in range(_WVECS):
                    xv = xin[h, pl.ds(w * 16, 16)]
                    t = jnp.minimum(jnp.maximum(xv, nsv), sv)
                    o1 = (t / sv + 1.0) / 2.0
                    o4 = o1 * 255.0 + offv
                    li = o4.astype(jnp.int32)
                    lof = li.astype(jnp.float32)
                    d = o4 - lof
                    ui = li + (d > 0.0).astype(jnp.int32)
                    lov = plsc.load_gather(tab_v, [li])
                    upv = plsc.load_gather(tab_v, [ui])
                    xout[h, pl.ds(w * 16, 16)] = d * lov + (1.0 - d) * upv

            pltpu.async_copy(xout, out_hbm.at[plane, pl.ds(h0, _BH)], osem)

            @pl.when(j < _BLOCKS_PER_W // 2 - 1)
            def _refill():
                start_in(k + 2, xin, isem)
        return 0

    lax.fori_loop(0, _BLOCKS_PER_W // 2, step, 0)

    p0, h0 = block_loc(_BLOCKS_PER_W - 2)
    p1, h1 = block_loc(_BLOCKS_PER_W - 1)
    pltpu.make_async_copy(
        xout0, out_hbm.at[p0, pl.ds(h0, _BH)], osem0).wait()
    pltpu.make_async_copy(
        xout1, out_hbm.at[p1, pl.ds(h1, _BH)], osem1).wait()


@jax.jit
def kernel(x, value_embeddings, scale):
    tab = value_embeddings.reshape(-1)
    sc = scale.reshape(-1)
    x3 = x.reshape(768, _H, _W)

    mesh = plsc.VectorSubcoreMesh(
        core_axis_name="c", subcore_axis_name="s", num_cores=_NC,
        num_subcores=_NS,
    )
    run = pl.kernel(
        _body,
        out_type=jax.ShapeDtypeStruct((768, _H, _W), jnp.float32),
        mesh=mesh,
        scratch_types=[
            pltpu.VMEM((tab.shape[0],), jnp.float32),
            pltpu.VMEM((_IN_CHANNELS,), jnp.float32),
            pltpu.VMEM((_BH, _W), jnp.float32),
            pltpu.VMEM((_BH, _W), jnp.float32),
            pltpu.VMEM((_BH, _W), jnp.float32),
            pltpu.VMEM((_BH, _W), jnp.float32),
            pltpu.SemaphoreType.DMA,
            pltpu.SemaphoreType.DMA,
            pltpu.SemaphoreType.DMA,
            pltpu.SemaphoreType.DMA,
        ],
        compiler_params=pltpu.CompilerParams(
            needs_layout_passes=False,
            use_tc_tiling_on_sc=True,
        ),
    )
    out = run(x3, tab, sc)
    return out.reshape(x.shape)


# speedup vs baseline: 2311.3235x; 1.0674x over previous
import functools

import jax
import jax.numpy as jnp
from jax import lax
from jax.experimental import pallas as pl
from jax.experimental.pallas import tpu as pltpu
from jax.experimental.pallas import tpu_sc as plsc

_NUM_EMB_PER_CH = 256
_IN_CHANNELS = 192
_MIN_SCALE = 2.5
_MAX_SCALE = 3.5

_H = 224
_W = 224
_BH = 56
_WVECS = _W // 16

_NC = 2
_NS = 16
_NW = _NC * _NS
_PLANES_PER_W = 768 // _NW
_BLOCKS_PER_W = _PLANES_PER_W * 4


def _body(x_hbm, tab_hbm, scale_hbm, out_hbm,
          tab_v, scale_v, xin0, xin1, xout0, xout1,
          isem0, isem1, osem0, osem1):
    wid = lax.axis_index("s") * _NC + lax.axis_index("c")

    pltpu.sync_copy(tab_hbm, tab_v)
    pltpu.sync_copy(scale_hbm, scale_v)

    def block_loc(k):
        plane = wid * _PLANES_PER_W + k // 4
        return plane, (k % 4) * _BH

    def start_in(k, buf, sem):
        plane, h0 = block_loc(k)
        pltpu.async_copy(x_hbm.at[plane, pl.ds(h0, _BH)], buf, sem)

    start_in(0, xin0, isem0)
    start_in(1, xin1, isem1)

    bufs = ((xin0, isem0, xout0, osem0), (xin1, isem1, xout1, osem1))

    def step(j, _):
        for b, (xin, isem, xout, osem) in enumerate(bufs):
            k = 2 * j + b
            plane, h0 = block_loc(k)
            ch = plane % _IN_CHANNELS

            pltpu.make_async_copy(
                x_hbm.at[plane, pl.ds(h0, _BH)], xin, isem).wait()

            @pl.when(j > 0)
            def _wait_out():
                pltpu.make_async_copy(
                    xout, out_hbm.at[plane, pl.ds(h0, _BH)], osem).wait()

            ch_vec = jnp.full((16,), ch, dtype=jnp.int32)
            sv = plsc.load_gather(scale_v, [ch_vec])
            sv = jnp.minimum(jnp.maximum(sv, _MIN_SCALE), _MAX_SCALE)
            nsv = -sv
            offv = jnp.full(
                (16,), ch * _NUM_EMB_PER_CH, dtype=jnp.int32
            ).astype(jnp.float32)

            @plsc.parallel_loop(0, _BH, unroll=2)
            def _row(h):
                for w in range(_WVECS):
                    xv = xin[h, pl.ds(w * 16, 16)]
                    t = jnp.minimum(jnp.maximum(xv, nsv), sv)
                    o4 = (t / sv + 1.0) * 127.5 + offv
                    li = o4.astype(jnp.int32)
                    lof = li.astype(jnp.float32)
                    d = o4 - lof
                    ui = li + (d > 0.0).astype(jnp.int32)
                    lov = plsc.load_gather(tab_v, [li])
                    upv = plsc.load_gather(tab_v, [ui])
                    xout[h, pl.ds(w * 16, 16)] = d * lov + (1.0 - d) * upv

            pltpu.async_copy(xout, out_hbm.at[plane, pl.ds(h0, _BH)], osem)

            @pl.when(j < _BLOCKS_PER_W // 2 - 1)
            def _refill():
                start_in(k + 2, xin, isem)
        return 0

    lax.fori_loop(0, _BLOCKS_PER_W // 2, step, 0)

    p0, h0 = block_loc(_BLOCKS_PER_W - 2)
    p1, h1 = block_loc(_BLOCKS_PER_W - 1)
    pltpu.make_async_copy(
        xout0, out_hbm.at[p0, pl.ds(h0, _BH)], osem0).wait()
    pltpu.make_async_copy(
        xout1, out_hbm.at[p1, pl.ds(h1, _BH)], osem1).wait()


@jax.jit
def kernel(x, value_embeddings, scale):
    tab = value_embeddings.reshape(-1)
    sc = scale.reshape(-1)
    x3 = x.reshape(768, _H, _W)

    mesh = plsc.VectorSubcoreMesh(
        core_axis_name="c", subcore_axis_name="s", num_cores=_NC,
        num_subcores=_NS,
    )
    run = pl.kernel(
        _body,
        out_type=jax.ShapeDtypeStruct((768, _H, _W), jnp.float32),
        mesh=mesh,
        scratch_types=[
            pltpu.VMEM((tab.shape[0],), jnp.float32),
            pltpu.VMEM((_IN_CHANNELS,), jnp.float32),
            pltpu.VMEM((_BH, _W), jnp.float32),
            pltpu.VMEM((_BH, _W), jnp.float32),
            pltpu.VMEM((_BH, _W), jnp.float32),
            pltpu.VMEM((_BH, _W), jnp.float32),
            pltpu.SemaphoreType.DMA,
            pltpu.SemaphoreType.DMA,
            pltpu.SemaphoreType.DMA,
            pltpu.SemaphoreType.DMA,
        ],
        compiler_params=pltpu.CompilerParams(
            needs_layout_passes=False,
            use_tc_tiling_on_sc=True,
        ),
    )
    out = run(x3, tab, sc)
    return out.reshape(x.shape)
